# Initial kernel scaffold; baseline (speedup 1.0000x reference)
import jax
import jax.numpy as jnp
from jax.experimental import pallas as pl


def kernel(hidden_states, gate_w, w_gate, w_up, w_down, shared_gate, shared_up, shared_down, e_score_correction_bias):
    raise NotImplementedError("write your pallas kernel here")


# speedup vs baseline: 1.2859x; 1.2859x over previous
import jax
import jax.numpy as jnp
from jax.experimental import pallas as pl
from jax.experimental.pallas import tpu as pltpu

T, D, E, K, FF, FFS = 64, 1024, 64, 2, 512, 1024


def _moe_step(bias_ref, x_ref, gate_w_ref, sg_ref, su_ref, sd_ref,
              wg_ref, wu_ref, wd_ref, out_ref, comb_ref):
    e = pl.program_id(0)

    @pl.when(e == 0)
    def _init():
        x = x_ref[...]
        logits = jnp.dot(x, gate_w_ref[...].T, preferred_element_type=jnp.float32)
        scores = jax.nn.sigmoid(logits)
        s_choice = scores + bias_ref[...]
        idx = jax.lax.broadcasted_iota(jnp.int32, (T, E), 1)
        m1 = jnp.max(s_choice, axis=1, keepdims=True)
        i1 = jnp.min(jnp.where(s_choice == m1, idx, E), axis=1, keepdims=True)
        masked = jnp.where(idx == i1, -jnp.inf, s_choice)
        m2 = jnp.max(masked, axis=1, keepdims=True)
        i2 = jnp.min(jnp.where(masked == m2, idx, E), axis=1, keepdims=True)
        w1 = jnp.sum(jnp.where(idx == i1, scores, 0.0), axis=1, keepdims=True)
        w2 = jnp.sum(jnp.where(idx == i2, scores, 0.0), axis=1, keepdims=True)
        denom = w1 + w2
        comb_ref[...] = (jnp.where(idx == i1, w1, 0.0)
                         + jnp.where(idx == i2, w2, 0.0)) / denom
        sg = jnp.dot(x, sg_ref[...], preferred_element_type=jnp.float32)
        su = jnp.dot(x, su_ref[...], preferred_element_type=jnp.float32)
        sh = jax.nn.silu(sg) * su
        out_ref[...] = jnp.dot(sh, sd_ref[...], preferred_element_type=jnp.float32)

    x = x_ref[...]
    g = jnp.dot(x, wg_ref[0], preferred_element_type=jnp.float32)
    u = jnp.dot(x, wu_ref[0], preferred_element_type=jnp.float32)
    h = jax.nn.silu(g) * u
    lane = jax.lax.broadcasted_iota(jnp.int32, (T, E), 1)
    ce = jnp.sum(jnp.where(lane == e, comb_ref[...], 0.0), axis=1, keepdims=True)
    out_ref[...] += jnp.dot(h * ce, wd_ref[0], preferred_element_type=jnp.float32)


def kernel(hidden_states, gate_w, w_gate, w_up, w_down, shared_gate,
           shared_up, shared_down, e_score_correction_bias):
    orig_shape = hidden_states.shape
    x = hidden_states.reshape(-1, orig_shape[-1])
    bias2d = e_score_correction_bias.reshape(1, E)

    out = pl.pallas_call(
        _moe_step,
        grid=(E,),
        in_specs=[
            pl.BlockSpec((1, E), lambda e: (0, 0)),
            pl.BlockSpec((T, D), lambda e: (0, 0)),
            pl.BlockSpec((E, D), lambda e: (0, 0)),
            pl.BlockSpec((D, FFS), lambda e: (0, 0)),
            pl.BlockSpec((D, FFS), lambda e: (0, 0)),
            pl.BlockSpec((FFS, D), lambda e: (0, 0)),
            pl.BlockSpec((1, D, FF), lambda e: (e, 0, 0)),
            pl.BlockSpec((1, D, FF), lambda e: (e, 0, 0)),
            pl.BlockSpec((1, FF, D), lambda e: (e, 0, 0)),
        ],
        out_specs=pl.BlockSpec((T, D), lambda e: (0, 0)),
        out_shape=jax.ShapeDtypeStruct((T, D), jnp.float32),
        scratch_shapes=[pltpu.VMEM((T, E), jnp.float32)],
    )(bias2d, x, gate_w, shared_gate, shared_up, shared_down,
      w_gate, w_up, w_down)
    return out.reshape(orig_shape)


# speedup vs baseline: 1.3181x; 1.0251x over previous
import jax
import jax.numpy as jnp
from jax.experimental import pallas as pl
from jax.experimental.pallas import tpu as pltpu

T, D, E, K, FF, FFS = 64, 1024, 64, 2, 512, 1024


def _router(bias_ref, x_ref, gate_w_ref, comb_ref, ids_ref):
    x = x_ref[...]
    logits = jnp.dot(x, gate_w_ref[...].T, preferred_element_type=jnp.float32)
    scores = jax.nn.sigmoid(logits)
    s_choice = scores + bias_ref[...]
    idx = jax.lax.broadcasted_iota(jnp.int32, (T, E), 1)
    m1 = jnp.max(s_choice, axis=1, keepdims=True)
    i1 = jnp.min(jnp.where(s_choice == m1, idx, E), axis=1, keepdims=True)
    masked = jnp.where(idx == i1, -jnp.inf, s_choice)
    m2 = jnp.max(masked, axis=1, keepdims=True)
    i2 = jnp.min(jnp.where(masked == m2, idx, E), axis=1, keepdims=True)
    w1 = jnp.sum(jnp.where(idx == i1, scores, 0.0), axis=1, keepdims=True)
    w2 = jnp.sum(jnp.where(idx == i2, scores, 0.0), axis=1, keepdims=True)
    denom = w1 + w2
    comb = (jnp.where(idx == i1, w1, 0.0) + jnp.where(idx == i2, w2, 0.0)) / denom
    comb_ref[...] = comb

    active_rows = jnp.sum(comb.T, axis=1, keepdims=True) > 0.0
    jj = jax.lax.broadcasted_iota(jnp.int32, (E, E), 0)
    ii = jax.lax.broadcasted_iota(jnp.int32, (E, E), 1)
    val = jnp.where((jj <= ii) & active_rows, jj, -1)
    eid_raw = jnp.max(val, axis=0, keepdims=True)
    jcol = jax.lax.broadcasted_iota(jnp.int32, (E, 1), 0)
    first_active = jnp.min(jnp.where(active_rows, jcol, E))
    ids_ref[...] = jnp.where(eid_raw < 0, first_active, eid_raw)


def _expert_step(ids_ref, bias_ref, comb_ref, x_ref, sg_ref, su_ref, sd_ref,
                 wg_ref, wu_ref, wd_ref, out_ref):
    i = pl.program_id(0)
    e = ids_ref[i]

    @pl.when(i == 0)
    def _init():
        x = x_ref[...]
        sg = jnp.dot(x, sg_ref[...], preferred_element_type=jnp.float32)
        su = jnp.dot(x, su_ref[...], preferred_element_type=jnp.float32)
        sh = jax.nn.silu(sg) * su
        out_ref[...] = jnp.dot(sh, sd_ref[...], preferred_element_type=jnp.float32)

    @pl.when(e == i)
    def _acc():
        x = x_ref[...]
        g = jnp.dot(x, wg_ref[0], preferred_element_type=jnp.float32)
        u = jnp.dot(x, wu_ref[0], preferred_element_type=jnp.float32)
        h = jax.nn.silu(g) * u
        lane = jax.lax.broadcasted_iota(jnp.int32, (T, E), 1)
        ce = jnp.sum(jnp.where(lane == i, comb_ref[...], 0.0), axis=1,
                     keepdims=True)
        out_ref[...] += jnp.dot(h * ce, wd_ref[0],
                                preferred_element_type=jnp.float32)


def kernel(hidden_states, gate_w, w_gate, w_up, w_down, shared_gate,
           shared_up, shared_down, e_score_correction_bias):
    orig_shape = hidden_states.shape
    x = hidden_states.reshape(-1, orig_shape[-1])
    bias2d = e_score_correction_bias.reshape(1, E)

    comb, ids = pl.pallas_call(
        _router,
        in_specs=[
            pl.BlockSpec((1, E), lambda: (0, 0)),
            pl.BlockSpec((T, D), lambda: (0, 0)),
            pl.BlockSpec((E, D), lambda: (0, 0)),
        ],
        out_specs=[
            pl.BlockSpec((T, E), lambda: (0, 0)),
            pl.BlockSpec((1, E), lambda: (0, 0)),
        ],
        out_shape=[
            jax.ShapeDtypeStruct((T, E), jnp.float32),
            jax.ShapeDtypeStruct((1, E), jnp.int32),
        ],
    )(bias2d, x, gate_w)

    grid_spec = pltpu.PrefetchScalarGridSpec(
        num_scalar_prefetch=1,
        grid=(E,),
        in_specs=[
            pl.BlockSpec((1, E), lambda i, ids: (0, 0)),
            pl.BlockSpec((T, E), lambda i, ids: (0, 0)),
            pl.BlockSpec((T, D), lambda i, ids: (0, 0)),
            pl.BlockSpec((D, FFS), lambda i, ids: (0, 0)),
            pl.BlockSpec((D, FFS), lambda i, ids: (0, 0)),
            pl.BlockSpec((FFS, D), lambda i, ids: (0, 0)),
            pl.BlockSpec((1, D, FF), lambda i, ids: (ids[i], 0, 0)),
            pl.BlockSpec((1, D, FF), lambda i, ids: (ids[i], 0, 0)),
            pl.BlockSpec((1, FF, D), lambda i, ids: (ids[i], 0, 0)),
        ],
        out_specs=pl.BlockSpec((T, D), lambda i, ids: (0, 0)),
    )
    out = pl.pallas_call(
        _expert_step,
        grid_spec=grid_spec,
        out_shape=jax.ShapeDtypeStruct((T, D), jnp.float32),
    )(ids.reshape(E), bias2d, comb, x, shared_gate, shared_up, shared_down,
      w_gate, w_up, w_down)
    return out.reshape(orig_shape)


# speedup vs baseline: 1.3358x; 1.0135x over previous
import jax
import jax.numpy as jnp
from jax.experimental import pallas as pl
from jax.experimental.pallas import tpu as pltpu

T, D, E, K, FF, FFS = 64, 1024, 64, 2, 512, 1024


def _router(bias_ref, x_ref, gate_w_ref, comb_ref, ids_ref):
    x = x_ref[...]
    logits = jnp.dot(x, gate_w_ref[...].T, preferred_element_type=jnp.float32)
    scores = jax.nn.sigmoid(logits)
    s_choice = scores + bias_ref[...]
    idx = jax.lax.broadcasted_iota(jnp.int32, (T, E), 1)
    m1 = jnp.max(s_choice, axis=1, keepdims=True)
    i1 = jnp.min(jnp.where(s_choice == m1, idx, E), axis=1, keepdims=True)
    masked = jnp.where(idx == i1, -jnp.inf, s_choice)
    m2 = jnp.max(masked, axis=1, keepdims=True)
    i2 = jnp.min(jnp.where(masked == m2, idx, E), axis=1, keepdims=True)
    w1 = jnp.sum(jnp.where(idx == i1, scores, 0.0), axis=1, keepdims=True)
    w2 = jnp.sum(jnp.where(idx == i2, scores, 0.0), axis=1, keepdims=True)
    denom = w1 + w2
    comb = (jnp.where(idx == i1, w1, 0.0) + jnp.where(idx == i2, w2, 0.0)) / denom
    comb_ref[...] = comb

    active_rows = jnp.sum(comb.T, axis=1, keepdims=True) > 0.0
    jj = jax.lax.broadcasted_iota(jnp.int32, (E, E), 0)
    ii = jax.lax.broadcasted_iota(jnp.int32, (E, E), 1)
    val = jnp.where((jj <= ii) & active_rows, jj, -1)
    eid_raw = jnp.max(val, axis=0, keepdims=True)
    jcol = jax.lax.broadcasted_iota(jnp.int32, (E, 1), 0)
    first_active = jnp.min(jnp.where(active_rows, jcol, E))
    ids_ref[...] = jnp.where(eid_raw < 0, first_active, eid_raw)


def _expert_step(ids_ref, bias_ref, comb_ref, x_ref, sg_ref, su_ref, sd_ref,
                 wg_ref, wu_ref, wd_ref, out_ref):
    i = pl.program_id(0)
    e = ids_ref[i]

    @pl.when(i == 0)
    def _init():
        xb = x_ref[...].astype(jnp.bfloat16)
        sg = jnp.dot(xb, sg_ref[...].astype(jnp.bfloat16),
                     preferred_element_type=jnp.float32)
        su = jnp.dot(xb, su_ref[...].astype(jnp.bfloat16),
                     preferred_element_type=jnp.float32)
        sh = jax.nn.silu(sg) * su
        out_ref[...] = jnp.dot(sh.astype(jnp.bfloat16),
                               sd_ref[...].astype(jnp.bfloat16),
                               preferred_element_type=jnp.float32)

    @pl.when(e == i)
    def _acc():
        xb = x_ref[...].astype(jnp.bfloat16)
        g = jnp.dot(xb, wg_ref[0].astype(jnp.bfloat16),
                    preferred_element_type=jnp.float32)
        u = jnp.dot(xb, wu_ref[0].astype(jnp.bfloat16),
                    preferred_element_type=jnp.float32)
        h = jax.nn.silu(g) * u
        lane = jax.lax.broadcasted_iota(jnp.int32, (T, E), 1)
        ce = jnp.sum(jnp.where(lane == i, comb_ref[...], 0.0), axis=1,
                     keepdims=True)
        out_ref[...] += jnp.dot((h * ce).astype(jnp.bfloat16),
                                wd_ref[0].astype(jnp.bfloat16),
                                preferred_element_type=jnp.float32)


def kernel(hidden_states, gate_w, w_gate, w_up, w_down, shared_gate,
           shared_up, shared_down, e_score_correction_bias):
    orig_shape = hidden_states.shape
    x = hidden_states.reshape(-1, orig_shape[-1])
    bias2d = e_score_correction_bias.reshape(1, E)

    comb, ids = pl.pallas_call(
        _router,
        in_specs=[
            pl.BlockSpec((1, E), lambda: (0, 0)),
            pl.BlockSpec((T, D), lambda: (0, 0)),
            pl.BlockSpec((E, D), lambda: (0, 0)),
        ],
        out_specs=[
            pl.BlockSpec((T, E), lambda: (0, 0)),
            pl.BlockSpec((1, E), lambda: (0, 0)),
        ],
        out_shape=[
            jax.ShapeDtypeStruct((T, E), jnp.float32),
            jax.ShapeDtypeStruct((1, E), jnp.int32),
        ],
    )(bias2d, x, gate_w)

    grid_spec = pltpu.PrefetchScalarGridSpec(
        num_scalar_prefetch=1,
        grid=(E,),
        in_specs=[
            pl.BlockSpec((1, E), lambda i, ids: (0, 0)),
            pl.BlockSpec((T, E), lambda i, ids: (0, 0)),
            pl.BlockSpec((T, D), lambda i, ids: (0, 0)),
            pl.BlockSpec((D, FFS), lambda i, ids: (0, 0)),
            pl.BlockSpec((D, FFS), lambda i, ids: (0, 0)),
            pl.BlockSpec((FFS, D), lambda i, ids: (0, 0)),
            pl.BlockSpec((1, D, FF), lambda i, ids: (ids[i], 0, 0)),
            pl.BlockSpec((1, D, FF), lambda i, ids: (ids[i], 0, 0)),
            pl.BlockSpec((1, FF, D), lambda i, ids: (ids[i], 0, 0)),
        ],
        out_specs=pl.BlockSpec((T, D), lambda i, ids: (0, 0)),
    )
    out = pl.pallas_call(
        _expert_step,
        grid_spec=grid_spec,
        out_shape=jax.ShapeDtypeStruct((T, D), jnp.float32),
    )(ids.reshape(E), bias2d, comb, x, shared_gate, shared_up, shared_down,
      w_gate, w_up, w_down)
    return out.reshape(orig_shape)
